# baseline (speedup 1.0000x reference)
import functools

import jax
import jax.numpy as jnp
from jax import lax
from jax.experimental import pallas as pl
from jax.experimental.pallas import tpu as pltpu
from jax.experimental.pallas import tpu_sc as plsc

N = 2048
DIM = 128
TDIM = 3 * DIM
ACCW = TDIM + DIM
NSUB = 16
NW = 32
CPW = N // NW
LANES = 16


def _gates(iou, fc):
    ig = jax.nn.sigmoid(iou[:, :DIM])
    og = jax.nn.sigmoid(iou[:, DIM:2 * DIM])
    ug = jnp.tanh(iou[:, 2 * DIM:TDIM])
    cc = ig * ug + fc
    hc = og * jnp.tanh(cc)
    return cc, hc


def _scatter_sel(lvl_v, pv_ref, r):
    jcol = jax.lax.broadcasted_iota(jnp.int32, (N, 1), 0)
    return jnp.logical_and(lvl_v == r, pv_ref[...] < jcol)


def _init_body(x_ref, pv_ref, wioux_ref, wfx_ref, wcat_ref,
               acc_ref, xf_ref, ewiou_ref, ewf_ref, ce_ref, h_ref, lvl_ref,
               tot_ref, gt_ref):
    ones_col = jnp.ones((N, 1), jnp.bfloat16)
    jrow = jax.lax.broadcasted_iota(jnp.int32, (N, DIM), 0)
    dnum = (((0,), (0,)), ((), ()))
    for blk in range(N // DIM):
        lane_ids = jax.lax.broadcasted_iota(jnp.int32, (N, DIM), 1) + blk * DIM
        eq = pv_ref[...] == lane_ids
        eqf = eq.astype(jnp.bfloat16)
        gtf = (eq & (jrow > lane_ids)).astype(jnp.bfloat16)
        tot_ref[pl.ds(blk * DIM, DIM), :] = jax.lax.dot_general(
            eqf, ones_col, dnum, preferred_element_type=jnp.float32)
        gt_ref[pl.ds(blk * DIM, DIM), :] = jax.lax.dot_general(
            gtf, ones_col, dnum, preferred_element_type=jnp.float32)
    hard_col = (gt_ref[...] > 0.0).astype(jnp.bfloat16)
    for blk in range(N // DIM):
        lane_ids = jax.lax.broadcasted_iota(jnp.int32, (N, DIM), 1) + blk * DIM
        eq = pv_ref[...] == lane_ids
        gtf = (eq & (jrow > lane_ids)).astype(jnp.bfloat16) * hard_col
        nh = jax.lax.dot_general(
            gtf, ones_col, dnum, preferred_element_type=jnp.float32)
        lvl_ref[pl.ds(blk * DIM, DIM), :] = nh.astype(jnp.int32)
    nhard = lvl_ref[...].astype(jnp.float32)
    is0 = gt_ref[...] == 0.0
    is1 = jnp.logical_and(jnp.logical_not(is0), nhard == 0.0)
    lvl_ref[...] = jnp.where(is0, 0, jnp.where(is1, 1, 2)).astype(jnp.int32)

    dn = (((1,), (1,)), ((), ()))
    acc_ref[...] = jax.lax.dot_general(
        x_ref[...], wioux_ref[...], dn, preferred_element_type=jnp.float32)
    xf_ref[...] = jax.lax.dot_general(
        x_ref[...], wfx_ref[...], dn, preferred_element_type=jnp.float32)

    cc, hc = _gates(acc_ref[...], jnp.float32(0.0))
    is_leaf = tot_ref[...] == 0.0
    xi = x_ref[...]
    cc = jnp.where(is_leaf, jnp.tanh(xi), cc)
    hc = jnp.where(is_leaf, xi, hc)

    lvl_v = lvl_ref[...]
    mask0 = lvl_v == 0
    sm0f = _scatter_sel(lvl_v, pv_ref, 0).astype(jnp.float32)
    h_ref[...] = hc * mask0.astype(jnp.float32)
    hw = jax.lax.dot_general(
        hc * sm0f, wcat_ref[...], dn, preferred_element_type=jnp.float32)
    ewiou_ref[...] = hw[:, :TDIM]
    ewf_ref[...] = hw[:, TDIM:]
    ce_ref[...] = cc * sm0f


def _mm_scatter(pr_ref, ewiou_ref, msgf_ref, acc_in_ref, acc_ref,
                fc_in=True):
    dm = (((1,), (0,)), ((), ()))
    ih = ewiou_ref[...].astype(jnp.bfloat16)
    il = (ewiou_ref[...] - ih.astype(jnp.float32)).astype(jnp.bfloat16)
    fh = msgf_ref[...].astype(jnp.bfloat16)
    fl = (msgf_ref[...] - fh.astype(jnp.float32)).astype(jnp.bfloat16)
    for blk in range(N // DIM):
        pid = jax.lax.broadcasted_iota(jnp.int32, (DIM, N), 0) + blk * DIM
        eqT = (pr_ref[...] == pid).astype(jnp.bfloat16)
        add_iou = (jax.lax.dot_general(
                       eqT, ih, dm, preferred_element_type=jnp.float32)
                   + jax.lax.dot_general(
                       eqT, il, dm, preferred_element_type=jnp.float32))
        add_fc = (jax.lax.dot_general(
                      eqT, fh, dm, preferred_element_type=jnp.float32)
                  + jax.lax.dot_general(
                      eqT, fl, dm, preferred_element_type=jnp.float32))
        acc_ref[pl.ds(blk * DIM, DIM), :TDIM] = (
            acc_in_ref[pl.ds(blk * DIM, DIM), :TDIM] + add_iou)
        if fc_in:
            acc_ref[pl.ds(blk * DIM, DIM), TDIM:] = (
                acc_in_ref[pl.ds(blk * DIM, DIM), TDIM:] + add_fc)
        else:
            acc_ref[pl.ds(blk * DIM, DIM), TDIM:] = add_fc


def _apply_round1_body(acc_in_ref, ewiou_ref, msgf_ref, lvl_ref,
                       pv_ref, pr_ref, h_in_ref, wcat_ref,
                       acc_ref, ewiou1_ref, ewf1_ref, ce_ref, h_ref):
    dn = (((1,), (1,)), ((), ()))
    _mm_scatter(pr_ref, ewiou_ref, msgf_ref, acc_in_ref, acc_ref,
                fc_in=False)

    cc, hc = _gates(acc_ref[:, :TDIM], acc_ref[:, TDIM:])
    lvl_v = lvl_ref[...]
    mask1 = lvl_v == 1
    sm1 = _scatter_sel(lvl_v, pv_ref, 1).astype(jnp.float32)
    h_ref[...] = jnp.where(mask1, hc, h_in_ref[...])
    hw = jax.lax.dot_general(
        hc * sm1, wcat_ref[...], dn, preferred_element_type=jnp.float32)
    ewiou1_ref[...] = hw[:, :TDIM]
    ewf1_ref[...] = hw[:, TDIM:]
    ce_ref[...] = cc * sm1


def _apply_residue_body(parent_smem, lvl_smem,
                        acc_in_ref, ewiou_ref, msgf_ref, xf_ref, h_in_ref,
                        pr_ref, wcat_ref, h_ref, acc_ref):
    dn = (((1,), (1,)), ((), ()))
    h_ref[...] = h_in_ref[...]
    _mm_scatter(pr_ref, ewiou_ref, msgf_ref, acc_in_ref, acc_ref)

    def step(t, _):
        idx = N - 1 - t

        @pl.when(lvl_smem[idx] == 2)
        def _():
            p = parent_smem[idx]
            row = acc_ref[pl.ds(idx, 1), :]
            cc, hc = _gates(row[:, :TDIM], row[:, TDIM:])
            h_ref[pl.ds(idx, 1), :] = hc

            @pl.when(p < idx)
            def _():
                hw = jax.lax.dot_general(
                    hc, wcat_ref[...], dn,
                    preferred_element_type=jnp.float32)
                prow = acc_ref[pl.ds(p, 1), :]
                xfp = xf_ref[pl.ds(p, 1), :]
                fmsg = jax.nn.sigmoid(xfp + hw[:, TDIM:]) * cc
                upd = jnp.concatenate([hw[:, :TDIM], fmsg], axis=1)
                acc_ref[pl.ds(p, 1), :] = prow + upd

        return 0

    jax.lax.fori_loop(0, N, step, 0)


def _sc_msg_body(xf_hbm, ewf_hbm, ce_hbm, parent_hbm, msgf_hbm,
                 pidx_v, ewf_v, ce_v, xfp_v, sem, sem2, sem3):
    w = lax.axis_index("c") * NSUB + lax.axis_index("s")
    base = w * CPW

    cp_ew = pltpu.async_copy(ewf_hbm.at[pl.ds(base, CPW)], ewf_v, sem2)
    cp_ce = pltpu.async_copy(ce_hbm.at[pl.ds(base, CPW)], ce_v, sem3)
    pltpu.sync_copy(parent_hbm.at[pl.ds(base, CPW)], pidx_v)
    pltpu.async_copy(xf_hbm.at[pidx_v], xfp_v, sem).wait()
    cp_ew.wait()
    cp_ce.wait()

    def edge(j, _):
        for k in range(DIM // LANES):
            hf = ewf_v[j, pl.ds(k * LANES, LANES)]
            xv = xfp_v[j, pl.ds(k * LANES, LANES)]
            sig = 1.0 / (1.0 + jnp.exp(-(xv + hf)))
            ewf_v[j, pl.ds(k * LANES, LANES)] = (
                sig * ce_v[j, pl.ds(k * LANES, LANES)])
        return 0

    lax.fori_loop(0, CPW, edge, 0)
    pltpu.sync_copy(ewf_v, msgf_hbm.at[pl.ds(base, CPW)])


def _sc_msgs(xf, ewf, ce, parent):
    mesh = plsc.VectorSubcoreMesh(core_axis_name="c", subcore_axis_name="s")
    f = functools.partial(
        pl.kernel, mesh=mesh,
        out_type=jax.ShapeDtypeStruct((N, DIM), jnp.float32),
        scratch_types=[
            pltpu.VMEM((CPW,), jnp.int32),
            pltpu.VMEM((CPW, DIM), jnp.float32),
            pltpu.VMEM((CPW, DIM), jnp.float32),
            pltpu.VMEM((CPW, DIM), jnp.float32),
            pltpu.SemaphoreType.DMA,
            pltpu.SemaphoreType.DMA,
            pltpu.SemaphoreType.DMA,
        ],
    )(_sc_msg_body)
    return f(xf, ewf, ce, parent)


def _vm(n=1):
    return [pl.BlockSpec(memory_space=pltpu.VMEM)] * n


def _sm(n=1):
    return [pl.BlockSpec(memory_space=pltpu.SMEM)] * n


def kernel(x, parent, depth, W_ioux, W_iouh, W_fx, W_fh):
    del depth
    parent = parent.astype(jnp.int32)
    pv = parent.reshape(N, 1)
    pr = parent.reshape(1, N)
    w_cat = jnp.concatenate([W_iouh, W_fh], axis=0)

    acc0, xf, ewiou0, ewf0, ce0, h0, lvl = pl.pallas_call(
        _init_body,
        out_shape=(
            jax.ShapeDtypeStruct((N, TDIM), jnp.float32),
            jax.ShapeDtypeStruct((N, DIM), jnp.float32),
            jax.ShapeDtypeStruct((N, TDIM), jnp.float32),
            jax.ShapeDtypeStruct((N, DIM), jnp.float32),
            jax.ShapeDtypeStruct((N, DIM), jnp.float32),
            jax.ShapeDtypeStruct((N, DIM), jnp.float32),
            jax.ShapeDtypeStruct((N, 1), jnp.int32),
        ),
        in_specs=_vm(5),
        out_specs=tuple(_vm(7)),
        scratch_shapes=[
            pltpu.VMEM((N, 1), jnp.float32),
            pltpu.VMEM((N, 1), jnp.float32),
        ],
    )(x, pv, W_ioux, W_fx, w_cat)

    msgf0 = _sc_msgs(xf, ewf0, ce0, parent)

    acc1, ewiou1, ewf1, ce1, h1 = pl.pallas_call(
        _apply_round1_body,
        out_shape=(
            jax.ShapeDtypeStruct((N, ACCW), jnp.float32),
            jax.ShapeDtypeStruct((N, TDIM), jnp.float32),
            jax.ShapeDtypeStruct((N, DIM), jnp.float32),
            jax.ShapeDtypeStruct((N, DIM), jnp.float32),
            jax.ShapeDtypeStruct((N, DIM), jnp.float32),
        ),
        in_specs=_vm(8),
        out_specs=tuple(_vm(5)),
    )(acc0, ewiou0, msgf0, lvl, pv, pr, h0, w_cat)

    msgf1 = _sc_msgs(xf, ewf1, ce1, parent)

    return pl.pallas_call(
        _apply_residue_body,
        out_shape=jax.ShapeDtypeStruct((N, DIM), jnp.float32),
        in_specs=_sm(2) + _vm(7),
        out_specs=pl.BlockSpec(memory_space=pltpu.VMEM),
        scratch_shapes=[pltpu.VMEM((N, ACCW), jnp.float32)],
    )(parent, lvl.reshape(N), acc1, ewiou1, msgf1, xf, h1, pr, w_cat)


# speedup vs baseline: 1.0831x; 1.0831x over previous
import functools

import jax
import jax.numpy as jnp
from jax import lax
from jax.experimental import pallas as pl
from jax.experimental.pallas import tpu as pltpu
from jax.experimental.pallas import tpu_sc as plsc

N = 2048
DIM = 128
TDIM = 3 * DIM
ACCW = TDIM + DIM
NSUB = 16
NW = 32
CPW = N // NW
LANES = 16


def _gates(iou, fc):
    ig = jax.nn.sigmoid(iou[:, :DIM])
    og = jax.nn.sigmoid(iou[:, DIM:2 * DIM])
    ug = jnp.tanh(iou[:, 2 * DIM:TDIM])
    cc = ig * ug + fc
    hc = og * jnp.tanh(cc)
    return cc, hc


def _scatter_sel(lvl_v, pv_ref, r):
    jcol = jax.lax.broadcasted_iota(jnp.int32, (N, 1), 0)
    return jnp.logical_and(lvl_v == r, pv_ref[...] < jcol)


def _init_body(x_ref, pv_ref, wioux_ref, wfx_ref, wcat_ref,
               acc_ref, xf_ref, ewiou_ref, ewf_ref, ce_ref, h_ref, lvl_ref,
               tot_ref, gt_ref):
    ones_col = jnp.ones((N, 1), jnp.bfloat16)
    jrow = jax.lax.broadcasted_iota(jnp.int32, (N, DIM), 0)
    dnum = (((0,), (0,)), ((), ()))
    for blk in range(N // DIM):
        lane_ids = jax.lax.broadcasted_iota(jnp.int32, (N, DIM), 1) + blk * DIM
        eq = pv_ref[...] == lane_ids
        eqf = eq.astype(jnp.bfloat16)
        gtf = (eq & (jrow > lane_ids)).astype(jnp.bfloat16)
        tot_ref[pl.ds(blk * DIM, DIM), :] = jax.lax.dot_general(
            eqf, ones_col, dnum, preferred_element_type=jnp.float32)
        gt_ref[pl.ds(blk * DIM, DIM), :] = jax.lax.dot_general(
            gtf, ones_col, dnum, preferred_element_type=jnp.float32)
    hard_col = (gt_ref[...] > 0.0).astype(jnp.bfloat16)
    for blk in range(N // DIM):
        lane_ids = jax.lax.broadcasted_iota(jnp.int32, (N, DIM), 1) + blk * DIM
        eq = pv_ref[...] == lane_ids
        gtf = (eq & (jrow > lane_ids)).astype(jnp.bfloat16) * hard_col
        nh = jax.lax.dot_general(
            gtf, ones_col, dnum, preferred_element_type=jnp.float32)
        lvl_ref[pl.ds(blk * DIM, DIM), :] = nh.astype(jnp.int32)
    nhard = lvl_ref[...].astype(jnp.float32)
    is0 = gt_ref[...] == 0.0
    is1 = jnp.logical_and(jnp.logical_not(is0), nhard == 0.0)
    lvl_ref[...] = jnp.where(is0, 0, jnp.where(is1, 1, 2)).astype(jnp.int32)

    dn = (((1,), (1,)), ((), ()))
    acc_ref[...] = jax.lax.dot_general(
        x_ref[...], wioux_ref[...], dn, preferred_element_type=jnp.float32)
    xf_ref[...] = jax.lax.dot_general(
        x_ref[...], wfx_ref[...], dn, preferred_element_type=jnp.float32)

    cc, hc = _gates(acc_ref[...], jnp.float32(0.0))
    is_leaf = tot_ref[...] == 0.0
    xi = x_ref[...]
    cc = jnp.where(is_leaf, jnp.tanh(xi), cc)
    hc = jnp.where(is_leaf, xi, hc)

    lvl_v = lvl_ref[...]
    mask0 = lvl_v == 0
    sm0f = _scatter_sel(lvl_v, pv_ref, 0).astype(jnp.float32)
    h_ref[...] = hc * mask0.astype(jnp.float32)
    hw = jax.lax.dot_general(
        hc * sm0f, wcat_ref[...], dn, preferred_element_type=jnp.float32)
    ewiou_ref[...] = hw[:, :TDIM]
    ewf_ref[...] = hw[:, TDIM:]
    ce_ref[...] = cc * sm0f


def _mm_scatter(pr_ref, ewiou_ref, msgf_ref, acc_in_ref, acc_ref,
                fc_in=True):
    dm = (((1,), (0,)), ((), ()))
    ih = ewiou_ref[...].astype(jnp.bfloat16)
    il = (ewiou_ref[...] - ih.astype(jnp.float32)).astype(jnp.bfloat16)
    fh = msgf_ref[...].astype(jnp.bfloat16)
    fl = (msgf_ref[...] - fh.astype(jnp.float32)).astype(jnp.bfloat16)
    for blk in range(N // DIM):
        pid = jax.lax.broadcasted_iota(jnp.int32, (DIM, N), 0) + blk * DIM
        eqT = (pr_ref[...] == pid).astype(jnp.bfloat16)
        add_iou = (jax.lax.dot_general(
                       eqT, ih, dm, preferred_element_type=jnp.float32)
                   + jax.lax.dot_general(
                       eqT, il, dm, preferred_element_type=jnp.float32))
        add_fc = (jax.lax.dot_general(
                      eqT, fh, dm, preferred_element_type=jnp.float32)
                  + jax.lax.dot_general(
                      eqT, fl, dm, preferred_element_type=jnp.float32))
        acc_ref[pl.ds(blk * DIM, DIM), :TDIM] = (
            acc_in_ref[pl.ds(blk * DIM, DIM), :TDIM] + add_iou)
        if fc_in:
            acc_ref[pl.ds(blk * DIM, DIM), TDIM:] = (
                acc_in_ref[pl.ds(blk * DIM, DIM), TDIM:] + add_fc)
        else:
            acc_ref[pl.ds(blk * DIM, DIM), TDIM:] = add_fc


def _apply_round1_body(lvl_smem, acc_in_ref, ewiou_ref, msgf_ref, lvl_ref,
                       pv_ref, pr_ref, h_in_ref, wcat_ref,
                       acc_ref, ewiou1_ref, ewf1_ref, ce_ref, h_ref,
                       rlist_ref, cnt_ref):
    dn = (((1,), (1,)), ((), ()))

    def build(t, c2):
        idx = N - 1 - t
        rlist_ref[c2] = idx
        return c2 + (lvl_smem[idx] == 2).astype(jnp.int32)

    c2 = jax.lax.fori_loop(0, N, build, jnp.int32(0))
    cnt_ref[0] = c2
    for k in range(1, 8):
        cnt_ref[k] = 0

    _mm_scatter(pr_ref, ewiou_ref, msgf_ref, acc_in_ref, acc_ref,
                fc_in=False)

    cc, hc = _gates(acc_ref[:, :TDIM], acc_ref[:, TDIM:])
    lvl_v = lvl_ref[...]
    mask1 = lvl_v == 1
    sm1 = _scatter_sel(lvl_v, pv_ref, 1).astype(jnp.float32)
    h_ref[...] = jnp.where(mask1, hc, h_in_ref[...])
    hw = jax.lax.dot_general(
        hc * sm1, wcat_ref[...], dn, preferred_element_type=jnp.float32)
    ewiou1_ref[...] = hw[:, :TDIM]
    ewf1_ref[...] = hw[:, TDIM:]
    ce_ref[...] = cc * sm1


def _apply_residue_body(parent_smem, rlist_smem, cnt_smem,
                        acc_in_ref, ewiou_ref, msgf_ref, xf_ref, h_in_ref,
                        pr_ref, wcat_ref, h_ref, acc_ref):
    dn = (((1,), (1,)), ((), ()))
    h_ref[...] = h_in_ref[...]
    _mm_scatter(pr_ref, ewiou_ref, msgf_ref, acc_in_ref, acc_ref)

    def step(t, _):
        idx = rlist_smem[t]
        p = parent_smem[idx]
        row = acc_ref[pl.ds(idx, 1), :]
        cc, hc = _gates(row[:, :TDIM], row[:, TDIM:])
        h_ref[pl.ds(idx, 1), :] = hc

        @pl.when(p < idx)
        def _():
            hw = jax.lax.dot_general(
                hc, wcat_ref[...], dn, preferred_element_type=jnp.float32)
            prow = acc_ref[pl.ds(p, 1), :]
            xfp = xf_ref[pl.ds(p, 1), :]
            fmsg = jax.nn.sigmoid(xfp + hw[:, TDIM:]) * cc
            upd = jnp.concatenate([hw[:, :TDIM], fmsg], axis=1)
            acc_ref[pl.ds(p, 1), :] = prow + upd

        return 0

    jax.lax.fori_loop(0, cnt_smem[0], step, 0)


def _sc_msg_body(xf_hbm, ewf_hbm, ce_hbm, parent_hbm, msgf_hbm,
                 pidx_v, ewf_v, ce_v, xfp_v, sem, sem2, sem3):
    w = lax.axis_index("c") * NSUB + lax.axis_index("s")
    base = w * CPW

    cp_ew = pltpu.async_copy(ewf_hbm.at[pl.ds(base, CPW)], ewf_v, sem2)
    cp_ce = pltpu.async_copy(ce_hbm.at[pl.ds(base, CPW)], ce_v, sem3)
    pltpu.sync_copy(parent_hbm.at[pl.ds(base, CPW)], pidx_v)
    pltpu.async_copy(xf_hbm.at[pidx_v], xfp_v, sem).wait()
    cp_ew.wait()
    cp_ce.wait()

    def edge(j, _):
        for k in range(DIM // LANES):
            hf = ewf_v[j, pl.ds(k * LANES, LANES)]
            xv = xfp_v[j, pl.ds(k * LANES, LANES)]
            sig = 1.0 / (1.0 + jnp.exp(-(xv + hf)))
            ewf_v[j, pl.ds(k * LANES, LANES)] = (
                sig * ce_v[j, pl.ds(k * LANES, LANES)])
        return 0

    lax.fori_loop(0, CPW, edge, 0)
    pltpu.sync_copy(ewf_v, msgf_hbm.at[pl.ds(base, CPW)])


def _sc_msgs(xf, ewf, ce, parent):
    mesh = plsc.VectorSubcoreMesh(core_axis_name="c", subcore_axis_name="s")
    f = functools.partial(
        pl.kernel, mesh=mesh,
        out_type=jax.ShapeDtypeStruct((N, DIM), jnp.float32),
        scratch_types=[
            pltpu.VMEM((CPW,), jnp.int32),
            pltpu.VMEM((CPW, DIM), jnp.float32),
            pltpu.VMEM((CPW, DIM), jnp.float32),
            pltpu.VMEM((CPW, DIM), jnp.float32),
            pltpu.SemaphoreType.DMA,
            pltpu.SemaphoreType.DMA,
            pltpu.SemaphoreType.DMA,
        ],
    )(_sc_msg_body)
    return f(xf, ewf, ce, parent)


def _vm(n=1):
    return [pl.BlockSpec(memory_space=pltpu.VMEM)] * n


def _sm(n=1):
    return [pl.BlockSpec(memory_space=pltpu.SMEM)] * n


def kernel(x, parent, depth, W_ioux, W_iouh, W_fx, W_fh):
    del depth
    parent = parent.astype(jnp.int32)
    pv = parent.reshape(N, 1)
    pr = parent.reshape(1, N)
    w_cat = jnp.concatenate([W_iouh, W_fh], axis=0)

    acc0, xf, ewiou0, ewf0, ce0, h0, lvl = pl.pallas_call(
        _init_body,
        out_shape=(
            jax.ShapeDtypeStruct((N, TDIM), jnp.float32),
            jax.ShapeDtypeStruct((N, DIM), jnp.float32),
            jax.ShapeDtypeStruct((N, TDIM), jnp.float32),
            jax.ShapeDtypeStruct((N, DIM), jnp.float32),
            jax.ShapeDtypeStruct((N, DIM), jnp.float32),
            jax.ShapeDtypeStruct((N, DIM), jnp.float32),
            jax.ShapeDtypeStruct((N, 1), jnp.int32),
        ),
        in_specs=_vm(5),
        out_specs=tuple(_vm(7)),
        scratch_shapes=[
            pltpu.VMEM((N, 1), jnp.float32),
            pltpu.VMEM((N, 1), jnp.float32),
        ],
    )(x, pv, W_ioux, W_fx, w_cat)

    msgf0 = _sc_msgs(xf, ewf0, ce0, parent)

    acc1, ewiou1, ewf1, ce1, h1, rlist, cnts = pl.pallas_call(
        _apply_round1_body,
        out_shape=(
            jax.ShapeDtypeStruct((N, ACCW), jnp.float32),
            jax.ShapeDtypeStruct((N, TDIM), jnp.float32),
            jax.ShapeDtypeStruct((N, DIM), jnp.float32),
            jax.ShapeDtypeStruct((N, DIM), jnp.float32),
            jax.ShapeDtypeStruct((N, DIM), jnp.float32),
            jax.ShapeDtypeStruct((N,), jnp.int32),
            jax.ShapeDtypeStruct((8,), jnp.int32),
        ),
        in_specs=_sm(1) + _vm(8),
        out_specs=tuple(_vm(5)) + tuple(_sm(2)),
    )(lvl.reshape(N), acc0, ewiou0, msgf0, lvl, pv, pr, h0, w_cat)

    msgf1 = _sc_msgs(xf, ewf1, ce1, parent)

    return pl.pallas_call(
        _apply_residue_body,
        out_shape=jax.ShapeDtypeStruct((N, DIM), jnp.float32),
        in_specs=_sm(3) + _vm(7),
        out_specs=pl.BlockSpec(memory_space=pltpu.VMEM),
        scratch_shapes=[pltpu.VMEM((N, ACCW), jnp.float32)],
    )(parent, rlist, cnts, acc1, ewiou1, msgf1, xf, h1, pr, w_cat)
